# baseline (speedup 1.0000x reference)
import jax
import jax.numpy as jnp
from jax.experimental import pallas as pl
from jax.experimental.pallas import tpu as pltpu

Q = 1024
D = 128
K = 100000
QBLK = 128
KBLK = 4096
CW = 256
NQ = Q // QBLK
NK = (K + KBLK - 1) // KBLK
KPAD = NK * KBLK
NCH = KBLK // CW
IMAX = 2**31 - 1


def _nn_kernel(q2_ref, qn_ref, kn_ref, k_ref, min_ref, idx_ref, sb_ref, sc_ref):
    kid = pl.program_id(0)
    i = pl.program_id(1)

    q2 = q2_ref[...]
    qn = qn_ref[...]
    sl = pl.ds(i * QBLK, QBLK)

    def window(best, argc):
        for c in range(NCH):
            kc = k_ref[pl.ds(c * CW, CW), :]
            qk2 = jax.lax.dot_general(
                q2, kc, (((1,), (1,)), ((), ())),
                preferred_element_type=jnp.float32)
            kn_c = kn_ref[pl.ds(c, 1), :]
            e = (qn + kn_c) + qk2
            take = e < best
            best = jnp.minimum(best, e)
            argc = jnp.where(take, kid * NCH + c, argc)
        return best, argc

    @pl.when(kid == 0)
    def _init():
        sb_ref[sl, :] = jnp.full((QBLK, CW), jnp.inf, dtype=jnp.float32)
        sc_ref[sl, :] = jnp.zeros((QBLK, CW), dtype=jnp.int32)

    best, argc = window(sb_ref[sl, :], sc_ref[sl, :])
    sb_ref[sl, :] = best
    sc_ref[sl, :] = argc

    @pl.when(kid == NK - 1)
    def _finish():
        best = sb_ref[sl, :]
        argc = sc_ref[sl, :]
        coli = jax.lax.broadcasted_iota(jnp.int32, (QBLK, CW), 1)
        gidx = argc * CW + coli
        rowmin = jnp.min(best, axis=1, keepdims=True)
        rowarg = jnp.min(jnp.where(best == rowmin, gidx, IMAX),
                         axis=1, keepdims=True)
        min_ref[...] = jnp.sqrt(jnp.maximum(rowmin, 1e-12))
        idx_ref[...] = rowarg


@jax.jit
def kernel(queries, keys):
    qn = jnp.sum(queries * queries, axis=1, keepdims=True)
    kn = jnp.sum(keys * keys, axis=1)
    q2 = queries * -2.0
    kn_p = jnp.concatenate(
        [kn, jnp.full((KPAD - K,), 1e12, jnp.float32)]).reshape(NK * NCH, CW)
    keys_p = jnp.concatenate(
        [keys, jnp.zeros((KPAD - K, D), jnp.float32)], axis=0)
    min_d, idx = pl.pallas_call(
        _nn_kernel,
        grid=(NK, NQ),
        in_specs=[
            pl.BlockSpec((QBLK, D), lambda k, i: (i, 0)),
            pl.BlockSpec((QBLK, 1), lambda k, i: (i, 0)),
            pl.BlockSpec((NCH, CW), lambda k, i: (k, 0)),
            pl.BlockSpec((KBLK, D), lambda k, i: (k, 0)),
        ],
        out_specs=[
            pl.BlockSpec((QBLK, 1), lambda k, i: (i, 0)),
            pl.BlockSpec((QBLK, 1), lambda k, i: (i, 0)),
        ],
        out_shape=[
            jax.ShapeDtypeStruct((Q, 1), jnp.float32),
            jax.ShapeDtypeStruct((Q, 1), jnp.int32),
        ],
        scratch_shapes=[
            pltpu.VMEM((Q, CW), jnp.float32),
            pltpu.VMEM((Q, CW), jnp.int32),
        ],
    )(q2, qn, kn_p, keys_p)
    return (min_d[:, 0], idx[:, 0])


# speedup vs baseline: 1.1327x; 1.1327x over previous
import jax
import jax.numpy as jnp
from jax.experimental import pallas as pl
from jax.experimental.pallas import tpu as pltpu

Q = 1024
D = 128
K = 100000
QBLK = 128
KBLK = 4096
CW = 256
NQ = Q // QBLK
NK = (K + KBLK - 1) // KBLK
KPAD = NK * KBLK
NCH = KBLK // CW
IMAX = 2**31 - 1


def _nn_kernel(q2_ref, qn_ref, kn_ref, k_ref, kt_ref,
               min_ref, idx_ref, sb_ref, sc_ref):
    kid = pl.program_id(0)
    i = pl.program_id(1)

    q2 = q2_ref[...]
    qn = qn_ref[...]
    sl = pl.ds(i * QBLK, QBLK)

    def window(src_ref, best, argc):
        for c in range(NCH):
            kc = src_ref[pl.ds(c * CW, CW), :]
            qk2 = jax.lax.dot_general(
                q2, kc, (((1,), (1,)), ((), ())),
                preferred_element_type=jnp.float32)
            kn_c = kn_ref[pl.ds(c, 1), :]
            e = (qn + kn_c) + qk2
            take = e < best
            best = jnp.minimum(best, e)
            argc = jnp.where(take, kid * NCH + c, argc)
        return best, argc

    @pl.when(kid == 0)
    def _init():
        sb_ref[sl, :] = jnp.full((QBLK, CW), jnp.inf, dtype=jnp.float32)
        sc_ref[sl, :] = jnp.zeros((QBLK, CW), dtype=jnp.int32)

    @pl.when(kid < NK - 1)
    def _main():
        best, argc = window(k_ref, sb_ref[sl, :], sc_ref[sl, :])
        sb_ref[sl, :] = best
        sc_ref[sl, :] = argc

    @pl.when(kid == NK - 1)
    def _finish():
        best, argc = window(kt_ref, sb_ref[sl, :], sc_ref[sl, :])
        coli = jax.lax.broadcasted_iota(jnp.int32, (QBLK, CW), 1)
        gidx = argc * CW + coli
        rowmin = jnp.min(best, axis=1, keepdims=True)
        rowarg = jnp.min(jnp.where(best == rowmin, gidx, IMAX),
                         axis=1, keepdims=True)
        min_ref[...] = jnp.sqrt(jnp.maximum(rowmin, 1e-12))
        idx_ref[...] = rowarg


@jax.jit
def kernel(queries, keys):
    qn = jnp.sum(queries * queries, axis=1, keepdims=True)
    kn = jnp.sum(keys * keys, axis=1)
    q2 = queries * -2.0
    kn_p = jnp.concatenate(
        [kn, jnp.full((KPAD - K,), 1e12, jnp.float32)]).reshape(NK * NCH, CW)
    kfull = (NK - 1) * KBLK
    ktail = jnp.zeros((KBLK, D), jnp.float32).at[:K - kfull].set(keys[kfull:])
    min_d, idx = pl.pallas_call(
        _nn_kernel,
        grid=(NK, NQ),
        in_specs=[
            pl.BlockSpec((QBLK, D), lambda k, i: (i, 0)),
            pl.BlockSpec((QBLK, 1), lambda k, i: (i, 0)),
            pl.BlockSpec((NCH, CW), lambda k, i: (k, 0)),
            pl.BlockSpec((KBLK, D),
                         lambda k, i: (jnp.minimum(k, NK - 2), 0)),
            pl.BlockSpec((KBLK, D), lambda k, i: (0, 0)),
        ],
        out_specs=[
            pl.BlockSpec((QBLK, 1), lambda k, i: (i, 0)),
            pl.BlockSpec((QBLK, 1), lambda k, i: (i, 0)),
        ],
        out_shape=[
            jax.ShapeDtypeStruct((Q, 1), jnp.float32),
            jax.ShapeDtypeStruct((Q, 1), jnp.int32),
        ],
        scratch_shapes=[
            pltpu.VMEM((Q, CW), jnp.float32),
            pltpu.VMEM((Q, CW), jnp.int32),
        ],
    )(q2, qn, kn_p, keys, ktail)
    return (min_d[:, 0], idx[:, 0])


# speedup vs baseline: 1.5141x; 1.3368x over previous
import jax
import jax.numpy as jnp
from jax.experimental import pallas as pl
from jax.experimental.pallas import tpu as pltpu

Q = 1024
D = 128
K = 100000
QBLK = 256
KBLK = 4096
CW = 256
NQ = Q // QBLK
NK = (K + KBLK - 1) // KBLK
KPAD = NK * KBLK
NCH = KBLK // CW
IMAX = 2**31 - 1


def _nn_kernel(q2_ref, qn_ref, kn_ref, k_ref, kt_ref,
               min_ref, idx_ref, sb_ref, sc_ref):
    kid = pl.program_id(0)
    i = pl.program_id(1)

    q2 = q2_ref[...]
    qn = qn_ref[...]
    sl = pl.ds(i * QBLK, QBLK)

    def window(src_ref, best, argc):
        for c in range(NCH):
            kc = src_ref[pl.ds(c * CW, CW), :]
            qk2 = jax.lax.dot_general(
                q2, kc, (((1,), (1,)), ((), ())),
                preferred_element_type=jnp.float32)
            kn_c = kn_ref[pl.ds(c, 1), :]
            e = (qn + kn_c) + qk2
            take = e < best
            best = jnp.minimum(best, e)
            argc = jnp.where(take, kid * NCH + c, argc)
        return best, argc

    @pl.when(kid == 0)
    def _init():
        sb_ref[sl, :] = jnp.full((QBLK, CW), jnp.inf, dtype=jnp.float32)
        sc_ref[sl, :] = jnp.zeros((QBLK, CW), dtype=jnp.int32)

    @pl.when(kid < NK - 1)
    def _main():
        best, argc = window(k_ref, sb_ref[sl, :], sc_ref[sl, :])
        sb_ref[sl, :] = best
        sc_ref[sl, :] = argc

    @pl.when(kid == NK - 1)
    def _finish():
        best, argc = window(kt_ref, sb_ref[sl, :], sc_ref[sl, :])
        coli = jax.lax.broadcasted_iota(jnp.int32, (QBLK, CW), 1)
        gidx = argc * CW + coli
        rowmin = jnp.min(best, axis=1, keepdims=True)
        rowarg = jnp.min(jnp.where(best == rowmin, gidx, IMAX),
                         axis=1, keepdims=True)
        min_ref[...] = jnp.sqrt(jnp.maximum(rowmin, 1e-12))
        idx_ref[...] = rowarg


@jax.jit
def kernel(queries, keys):
    qn = jnp.sum(queries * queries, axis=1, keepdims=True)
    kn = jnp.sum(keys * keys, axis=1)
    q2 = queries * -2.0
    kn_p = jnp.concatenate(
        [kn, jnp.full((KPAD - K,), 1e12, jnp.float32)]).reshape(NK * NCH, CW)
    kfull = (NK - 1) * KBLK
    ktail = jnp.zeros((KBLK, D), jnp.float32).at[:K - kfull].set(keys[kfull:])
    min_d, idx = pl.pallas_call(
        _nn_kernel,
        grid=(NK, NQ),
        in_specs=[
            pl.BlockSpec((QBLK, D), lambda k, i: (i, 0)),
            pl.BlockSpec((QBLK, 1), lambda k, i: (i, 0)),
            pl.BlockSpec((NCH, CW), lambda k, i: (k, 0)),
            pl.BlockSpec((KBLK, D),
                         lambda k, i: (jnp.minimum(k, NK - 2), 0)),
            pl.BlockSpec((KBLK, D), lambda k, i: (0, 0)),
        ],
        out_specs=[
            pl.BlockSpec((QBLK, 1), lambda k, i: (i, 0)),
            pl.BlockSpec((QBLK, 1), lambda k, i: (i, 0)),
        ],
        out_shape=[
            jax.ShapeDtypeStruct((Q, 1), jnp.float32),
            jax.ShapeDtypeStruct((Q, 1), jnp.int32),
        ],
        scratch_shapes=[
            pltpu.VMEM((Q, CW), jnp.float32),
            pltpu.VMEM((Q, CW), jnp.int32),
        ],
    )(q2, qn, kn_p, keys, ktail)
    return (min_d[:, 0], idx[:, 0])


# speedup vs baseline: 1.8412x; 1.2160x over previous
import jax
import jax.numpy as jnp
from jax.experimental import pallas as pl
from jax.experimental.pallas import tpu as pltpu

Q = 1024
D = 128
K = 100000
QBLK = 512
KBLK = 4096
CW = 256
NQ = Q // QBLK
NK = (K + KBLK - 1) // KBLK
KPAD = NK * KBLK
NCH = KBLK // CW
IMAX = 2**31 - 1


def _nn_kernel(q2_ref, qn_ref, kn_ref, k_ref, kt_ref,
               min_ref, idx_ref, sb_ref, sc_ref):
    kid = pl.program_id(0)
    i = pl.program_id(1)

    q2 = q2_ref[...]
    qn = qn_ref[...]
    sl = pl.ds(i * QBLK, QBLK)

    def window(src_ref, best, argc):
        for c in range(NCH):
            kc = src_ref[pl.ds(c * CW, CW), :]
            qk2 = jax.lax.dot_general(
                q2, kc, (((1,), (1,)), ((), ())),
                preferred_element_type=jnp.float32)
            kn_c = kn_ref[pl.ds(c, 1), :]
            e = (qn + kn_c) + qk2
            take = e < best
            best = jnp.minimum(best, e)
            argc = jnp.where(take, kid * NCH + c, argc)
        return best, argc

    @pl.when(kid == 0)
    def _init():
        sb_ref[sl, :] = jnp.full((QBLK, CW), jnp.inf, dtype=jnp.float32)
        sc_ref[sl, :] = jnp.zeros((QBLK, CW), dtype=jnp.int32)

    @pl.when(kid < NK - 1)
    def _main():
        best, argc = window(k_ref, sb_ref[sl, :], sc_ref[sl, :])
        sb_ref[sl, :] = best
        sc_ref[sl, :] = argc

    @pl.when(kid == NK - 1)
    def _finish():
        best, argc = window(kt_ref, sb_ref[sl, :], sc_ref[sl, :])
        coli = jax.lax.broadcasted_iota(jnp.int32, (QBLK, CW), 1)
        gidx = argc * CW + coli
        rowmin = jnp.min(best, axis=1, keepdims=True)
        rowarg = jnp.min(jnp.where(best == rowmin, gidx, IMAX),
                         axis=1, keepdims=True)
        min_ref[...] = jnp.sqrt(jnp.maximum(rowmin, 1e-12))
        idx_ref[...] = rowarg


@jax.jit
def kernel(queries, keys):
    qn = jnp.sum(queries * queries, axis=1, keepdims=True)
    kn = jnp.sum(keys * keys, axis=1)
    q2 = queries * -2.0
    kn_p = jnp.concatenate(
        [kn, jnp.full((KPAD - K,), 1e12, jnp.float32)]).reshape(NK * NCH, CW)
    kfull = (NK - 1) * KBLK
    ktail = jnp.zeros((KBLK, D), jnp.float32).at[:K - kfull].set(keys[kfull:])
    min_d, idx = pl.pallas_call(
        _nn_kernel,
        grid=(NK, NQ),
        in_specs=[
            pl.BlockSpec((QBLK, D), lambda k, i: (i, 0)),
            pl.BlockSpec((QBLK, 1), lambda k, i: (i, 0)),
            pl.BlockSpec((NCH, CW), lambda k, i: (k, 0)),
            pl.BlockSpec((KBLK, D),
                         lambda k, i: (jnp.minimum(k, NK - 2), 0)),
            pl.BlockSpec((KBLK, D), lambda k, i: (0, 0)),
        ],
        out_specs=[
            pl.BlockSpec((QBLK, 1), lambda k, i: (i, 0)),
            pl.BlockSpec((QBLK, 1), lambda k, i: (i, 0)),
        ],
        out_shape=[
            jax.ShapeDtypeStruct((Q, 1), jnp.float32),
            jax.ShapeDtypeStruct((Q, 1), jnp.int32),
        ],
        scratch_shapes=[
            pltpu.VMEM((Q, CW), jnp.float32),
            pltpu.VMEM((Q, CW), jnp.int32),
        ],
    )(q2, qn, kn_p, keys, ktail)
    return (min_d[:, 0], idx[:, 0])


# speedup vs baseline: 1.9533x; 1.0609x over previous
import jax
import jax.numpy as jnp
from jax.experimental import pallas as pl
from jax.experimental.pallas import tpu as pltpu

Q = 1024
D = 128
K = 100000
QBLK = 1024
KBLK = 4096
CW = 256
NQ = Q // QBLK
NK = (K + KBLK - 1) // KBLK
KPAD = NK * KBLK
NCH = KBLK // CW
IMAX = 2**31 - 1


def _nn_kernel(q2_ref, qn_ref, kn_ref, k_ref, kt_ref,
               min_ref, idx_ref, sb_ref, sc_ref):
    kid = pl.program_id(0)
    i = pl.program_id(1)

    q2 = q2_ref[...]
    qn = qn_ref[...]
    sl = pl.ds(i * QBLK, QBLK)

    def window(src_ref, best, argc):
        for c in range(NCH):
            kc = src_ref[pl.ds(c * CW, CW), :]
            qk2 = jax.lax.dot_general(
                q2, kc, (((1,), (1,)), ((), ())),
                preferred_element_type=jnp.float32)
            kn_c = kn_ref[pl.ds(c, 1), :]
            e = (qn + kn_c) + qk2
            take = e < best
            best = jnp.minimum(best, e)
            argc = jnp.where(take, kid * NCH + c, argc)
        return best, argc

    @pl.when(kid == 0)
    def _init():
        sb_ref[sl, :] = jnp.full((QBLK, CW), jnp.inf, dtype=jnp.float32)
        sc_ref[sl, :] = jnp.zeros((QBLK, CW), dtype=jnp.int32)

    @pl.when(kid < NK - 1)
    def _main():
        best, argc = window(k_ref, sb_ref[sl, :], sc_ref[sl, :])
        sb_ref[sl, :] = best
        sc_ref[sl, :] = argc

    @pl.when(kid == NK - 1)
    def _finish():
        best, argc = window(kt_ref, sb_ref[sl, :], sc_ref[sl, :])
        coli = jax.lax.broadcasted_iota(jnp.int32, (QBLK, CW), 1)
        gidx = argc * CW + coli
        rowmin = jnp.min(best, axis=1, keepdims=True)
        rowarg = jnp.min(jnp.where(best == rowmin, gidx, IMAX),
                         axis=1, keepdims=True)
        min_ref[...] = jnp.sqrt(jnp.maximum(rowmin, 1e-12))
        idx_ref[...] = rowarg


@jax.jit
def kernel(queries, keys):
    qn = jnp.sum(queries * queries, axis=1, keepdims=True)
    kn = jnp.sum(keys * keys, axis=1)
    q2 = queries * -2.0
    kn_p = jnp.concatenate(
        [kn, jnp.full((KPAD - K,), 1e12, jnp.float32)]).reshape(NK * NCH, CW)
    kfull = (NK - 1) * KBLK
    ktail = jnp.zeros((KBLK, D), jnp.float32).at[:K - kfull].set(keys[kfull:])
    min_d, idx = pl.pallas_call(
        _nn_kernel,
        grid=(NK, NQ),
        in_specs=[
            pl.BlockSpec((QBLK, D), lambda k, i: (i, 0)),
            pl.BlockSpec((QBLK, 1), lambda k, i: (i, 0)),
            pl.BlockSpec((NCH, CW), lambda k, i: (k, 0)),
            pl.BlockSpec((KBLK, D),
                         lambda k, i: (jnp.minimum(k, NK - 2), 0)),
            pl.BlockSpec((KBLK, D), lambda k, i: (0, 0)),
        ],
        out_specs=[
            pl.BlockSpec((QBLK, 1), lambda k, i: (i, 0)),
            pl.BlockSpec((QBLK, 1), lambda k, i: (i, 0)),
        ],
        out_shape=[
            jax.ShapeDtypeStruct((Q, 1), jnp.float32),
            jax.ShapeDtypeStruct((Q, 1), jnp.int32),
        ],
        scratch_shapes=[
            pltpu.VMEM((Q, CW), jnp.float32),
            pltpu.VMEM((Q, CW), jnp.int32),
        ],
    )(q2, qn, kn_p, keys, ktail)
    return (min_d[:, 0], idx[:, 0])


# speedup vs baseline: 2.0578x; 1.0535x over previous
import jax
import jax.numpy as jnp
from jax.experimental import pallas as pl
from jax.experimental.pallas import tpu as pltpu

Q = 1024
D = 128
K = 100000
KBLK = 4096
CW = 256
TQ = 256
NT = Q // TQ
NK = (K + KBLK - 1) // KBLK
NCH = KBLK // CW
IMAX = 2**31 - 1


def _nn_kernel(q2_ref, qn_ref, kn_ref, k_ref, kt_ref,
               min_ref, idx_ref, sb_ref, sc_ref):
    kid = pl.program_id(0)

    def window(src_ref):
        for t in range(NT):
            ts = pl.ds(t * TQ, TQ)
            q2 = q2_ref[ts, :]
            qn = qn_ref[ts, :]
            best = sb_ref[ts, :]
            argc = sc_ref[ts, :]
            for c in range(NCH):
                kc = src_ref[pl.ds(c * CW, CW), :]
                qk2 = jax.lax.dot_general(
                    q2, kc, (((1,), (1,)), ((), ())),
                    preferred_element_type=jnp.float32)
                kn_c = kn_ref[pl.ds(c, 1), :]
                e = (qn + kn_c) + qk2
                take = e < best
                best = jnp.minimum(best, e)
                argc = jnp.where(take, kid * NCH + c, argc)
            sb_ref[ts, :] = best
            sc_ref[ts, :] = argc

    @pl.when(kid == 0)
    def _init():
        sb_ref[...] = jnp.full((Q, CW), jnp.inf, dtype=jnp.float32)
        sc_ref[...] = jnp.zeros((Q, CW), dtype=jnp.int32)

    @pl.when(kid < NK - 1)
    def _main():
        window(k_ref)

    @pl.when(kid == NK - 1)
    def _finish():
        window(kt_ref)
        best = sb_ref[...]
        argc = sc_ref[...]
        coli = jax.lax.broadcasted_iota(jnp.int32, (Q, CW), 1)
        gidx = argc * CW + coli
        rowmin = jnp.min(best, axis=1, keepdims=True)
        rowarg = jnp.min(jnp.where(best == rowmin, gidx, IMAX),
                         axis=1, keepdims=True)
        min_ref[...] = jnp.sqrt(jnp.maximum(rowmin, 1e-12))
        idx_ref[...] = rowarg


@jax.jit
def kernel(queries, keys):
    qn = jnp.sum(queries * queries, axis=1, keepdims=True)
    kn = jnp.sum(keys * keys, axis=1)
    q2 = queries * -2.0
    kn_p = jnp.concatenate(
        [kn, jnp.full((NK * KBLK - K,), 1e12, jnp.float32)]
    ).reshape(NK * NCH, CW)
    kfull = (NK - 1) * KBLK
    ktail = jnp.zeros((KBLK, D), jnp.float32).at[:K - kfull].set(keys[kfull:])
    min_d, idx = pl.pallas_call(
        _nn_kernel,
        grid=(NK,),
        in_specs=[
            pl.BlockSpec((Q, D), lambda k: (0, 0)),
            pl.BlockSpec((Q, 1), lambda k: (0, 0)),
            pl.BlockSpec((NCH, CW), lambda k: (k, 0)),
            pl.BlockSpec((KBLK, D), lambda k: (jnp.minimum(k, NK - 2), 0)),
            pl.BlockSpec((KBLK, D), lambda k: (0, 0)),
        ],
        out_specs=[
            pl.BlockSpec((Q, 1), lambda k: (0, 0)),
            pl.BlockSpec((Q, 1), lambda k: (0, 0)),
        ],
        out_shape=[
            jax.ShapeDtypeStruct((Q, 1), jnp.float32),
            jax.ShapeDtypeStruct((Q, 1), jnp.int32),
        ],
        scratch_shapes=[
            pltpu.VMEM((Q, CW), jnp.float32),
            pltpu.VMEM((Q, CW), jnp.int32),
        ],
    )(q2, qn, kn_p, keys, ktail)
    return (min_d[:, 0], idx[:, 0])
